# Initial kernel scaffold; baseline (speedup 1.0000x reference)
import jax
import jax.numpy as jnp
from jax.experimental import pallas as pl


def kernel(inputs, gru_wi0, gru_wh0, gru_bi0, gru_bh0, gru_wi1, gru_wh1, gru_bi1, gru_bh1, gru_wi2, gru_wh2, gru_bi2, gru_bh2, ang_w, ang_b, att_wk, att_bk, att_wg, att_bg, att_wa, att_wv, att_bv):
    raise NotImplementedError("write your pallas kernel here")


# speedup vs baseline: 10.0951x; 10.0951x over previous
import jax
import jax.numpy as jnp
from jax.experimental import pallas as pl
from jax.experimental.pallas import tpu as pltpu

_B, _L, _DIN = 4, 128, 128
_H = 512
_HEADS, _A = 8, 64
_NH = 3 * _H
_F = _H + _DIN + 9
_FPAD = 656

_INTERPRET = False


def _gru(x, s, wi, wh, bi, bh):
    gi = jnp.dot(x, wi, preferred_element_type=jnp.float32) + bi
    gh = jnp.dot(s, wh, preferred_element_type=jnp.float32) + bh
    r = jax.nn.sigmoid(gi[:, :_H] + gh[:, :_H])
    z = jax.nn.sigmoid(gi[:, _H:2 * _H] + gh[:, _H:2 * _H])
    n = jnp.tanh(gi[:, 2 * _H:] + r * gh[:, 2 * _H:])
    return (1.0 - z) * n + z * s


def _kern(inputs_ref, wi0, wh0, bi0, bh0, wi1, wh1, bi1, bh1, wi2, wh2, bi2,
          bh2, angw, angb, watt, batt, wabd, ebd, pos_ref, ang_ref, feat_ref,
          tmp_ref, kg_ref, sm_ref):
    feat_ref[...] = jnp.zeros((_B, _L, _FPAD), jnp.float32)
    pos_ref[...] = jnp.zeros((_B, _L, 9), jnp.float32)

    def step(idx, carry):
        s0, s1, s2, context, prev9 = carry
        inp = inputs_ref[:, pl.ds(idx, 1), :].reshape(_B, _DIN)
        x = jnp.concatenate([inp, context], axis=1)
        h = _gru(x, s0, wi0[...], wh0[...], bi0[...], bh0[...])
        tmp_ref[:, 0, :] = h
        h0 = tmp_ref[:, 0, :]
        h = _gru(h0, s1, wi1[...], wh1[...], bi1[...], bh1[...])
        tmp_ref[:, 1, :] = h
        h1 = tmp_ref[:, 1, :]
        h = _gru(h1, s2, wi2[...], wh2[...], bi2[...], bh2[...])
        tmp_ref[:, 2, :] = h
        h2 = tmp_ref[:, 2, :]

        pre = jnp.dot(h2, angw[...], preferred_element_type=jnp.float32) \
            + angb[...]
        angle = jnp.arctan2(pre[:, 0:3], pre[:, 3:6])
        c = jnp.cos(angle)
        s = jnp.sin(angle)
        cs = c * s
        n3 = jnp.sqrt(c * c + s * s + cs * cs) + 1e-6
        parts = []
        for j in range(3):
            nj = n3[:, j:j + 1]
            parts += [c[:, j:j + 1] / nj, s[:, j:j + 1] / nj,
                      cs[:, j:j + 1] / nj]
        tert9 = prev9 + 1.5 * jnp.concatenate(parts, axis=1)
        pos_ref[:, pl.ds(idx, 1), :] = tert9[:, None, :]
        ang_ref[:, pl.ds(idx, 1), :] = angle[:, None, :]

        row = jnp.concatenate(
            [h2, inp, s, c, jnp.zeros((_B, _FPAD - _F + 3), jnp.float32)],
            axis=1)
        feat_ref[:, pl.ds(idx, 1), :] = row[:, None, :]

        hist = pos_ref[...]
        diff = hist - tert9[:, None, :]
        sq = diff * diff
        tpos = jax.lax.broadcasted_iota(jnp.int32, (_B, _L), 1)
        valid = tpos <= idx
        cds = []
        for a in range(3):
            d2 = sq[:, :, 3 * a] + sq[:, :, 3 * a + 1] + sq[:, :, 3 * a + 2]
            cds.append(jnp.sqrt(d2 + 1e-12))
        ca = cds[1]
        cdcols = jnp.concatenate(
            [jnp.where(valid, cd, 0.0)[:, :, None] * 0.125 for cd in cds],
            axis=2)
        feat_ref[:, :, _F - 3:_F] = cdcols

        z = jnp.dot(feat_ref[...].reshape(_B * _L, _FPAD), watt[...],
                    preferred_element_type=jnp.float32)
        z = (z + batt[...]).reshape(_B, _L, _NH)
        k = jnp.tanh(z[:, :, :_H])
        g = jax.nn.sigmoid(z[:, :, _H:2 * _H])
        v = z[:, :, 2 * _H:]

        kg_ref[...] = (k * g).reshape(_B * _L, _H).astype(
            jnp.bfloat16).astype(jnp.float32)
        kg = kg_ref[...]
        logits = jnp.dot(kg, wabd[...], preferred_element_type=jnp.float32)
        logits = logits.reshape(_B, _L, _HEADS)
        ca3 = jax.lax.broadcast_in_dim(ca, (_B, _L, _HEADS), (0, 1))
        valid3 = jax.lax.broadcasted_iota(jnp.int32, (_B, _L, _HEADS), 1) \
            <= idx
        m3 = valid3 & (ca3 < 8.0)
        logits = jnp.where(m3, logits, -1e9)
        mx = jnp.max(logits, axis=1, keepdims=True)
        e = jnp.exp(logits - jax.lax.stop_gradient(mx))
        sm_ref[...] = (e / jnp.sum(e, axis=1, keepdims=True)).reshape(
            _B * _L, _HEADS)
        sm = sm_ref[...]
        w512 = jnp.dot(sm, ebd[...], preferred_element_type=jnp.float32)
        vq = v.astype(jnp.bfloat16).astype(jnp.float32)
        ctx = jnp.sum(w512.reshape(_B, _L, _H) * vq, axis=1)
        tmp_ref[:, 3, :] = jnp.where(idx > 0, ctx, jnp.zeros_like(ctx))
        context_new = tmp_ref[:, 3, :]
        return (h0, h1, h2, context_new, tert9)

    init = (jnp.zeros((_B, _H), jnp.float32),
            jnp.zeros((_B, _H), jnp.float32),
            jnp.zeros((_B, _H), jnp.float32),
            jnp.zeros((_B, _H), jnp.float32),
            jnp.zeros((_B, 9), jnp.float32))
    jax.lax.fori_loop(0, _L, step, init)


def kernel(inputs, gru_wi0, gru_wh0, gru_bi0, gru_bh0, gru_wi1, gru_wh1,
           gru_bi1, gru_bh1, gru_wi2, gru_wh2, gru_bi2, gru_bh2, ang_w,
           ang_b, att_wk, att_bk, att_wg, att_bg, att_wa, att_wv, att_bv):
    f32 = jnp.float32
    watt = jnp.concatenate(
        [att_wk, att_wg, att_wv], axis=1)
    watt = jnp.concatenate(
        [watt, jnp.zeros((_FPAD - _F, _NH), f32)], axis=0)
    batt = jnp.concatenate([att_bk, att_bg, att_bv])[None, :]
    wabd = (att_wa[:, :, None]
            * jnp.eye(_HEADS, dtype=f32)[:, None, :]).reshape(
                _HEADS * _A, _HEADS)
    wabd = wabd.astype(jnp.bfloat16).astype(f32)
    ebd = jnp.repeat(jnp.eye(_HEADS, dtype=f32), _A, axis=1)
    angw = jnp.concatenate([ang_w, jnp.zeros((_H, 2), f32)], axis=1)
    angb = jnp.concatenate([ang_b, jnp.zeros((2,), f32)])[None, :]

    pos9, ang = pl.pallas_call(
        _kern,
        out_shape=[jax.ShapeDtypeStruct((_B, _L, 9), f32),
                   jax.ShapeDtypeStruct((_B, _L, 3), f32)],
        scratch_shapes=[pltpu.VMEM((_B, _L, _FPAD), f32),
                        pltpu.VMEM((_B, 4, _H), f32),
                        pltpu.VMEM((_B * _L, _H), f32),
                        pltpu.VMEM((_B * _L, _HEADS), f32)],
        interpret=_INTERPRET,
    )(inputs, gru_wi0, gru_wh0, gru_bi0[None], gru_bh0[None], gru_wi1,
      gru_wh1, gru_bi1[None], gru_bh1[None], gru_wi2, gru_wh2, gru_bi2[None],
      gru_bh2[None], angw, angb, watt, batt, wabd, ebd)
    return (pos9.reshape(_B, _L, 3, 3), ang)
